# baseline (speedup 1.0000x reference)
import jax
import jax.numpy as jnp
from jax import lax
from jax.experimental import pallas as pl
from jax.experimental.pallas import tpu as pltpu
from jax.experimental.pallas import tpu_sc as plsc

N_USERS = 25000
N_ITEMS = 25000
N_NODES = 50000
NNZ = 800000
DIM = 64
HALF = 32
TEXT_DIM = 384
N_ITEM_BATCH = 12288

N_TILES = 16
N_WORKERS = 32
SCHUNK = 128
CHUNK = 512
N_CHUNKS = 99
LAST = N_CHUNKS - 1
EDGES_PER_TILE = CHUNK * N_CHUNKS
NNZ_PAD = N_TILES * EDGES_PER_TILE
ROW2_ROWS = NNZ_PAD // SCHUNK

X_PAD_HALF = 50048
ROWS_PER_TILE = 3128
ZCHUNK = 136
N_ZCHUNK = 23
TEXT_PER_W = N_ITEM_BATCH // N_WORKERS
TCHUNK = 128


def _text_sc_body(tf, itemsr, tfg, tidx, tbuf0, tbuf1, sem0, sem1):
    c = lax.axis_index("c")
    s = lax.axis_index("s")
    wid = s * 2 + c
    tbase = wid * TEXT_PER_W
    pltpu.sync_copy(itemsr.at[pl.ds(tbase, TEXT_PER_W)], tidx)
    bufs = (tbuf0, tbuf1)
    sems = (sem0, sem1)
    descs = [
        pltpu.async_copy(tf.at[tidx.at[pl.ds(k * TCHUNK, TCHUNK)]],
                         bufs[k % 2], sems[k % 2])
        for k in range(2)
    ]
    for k in range(TEXT_PER_W // TCHUNK):
        descs[k].wait()
        pltpu.sync_copy(bufs[k % 2], tfg.at[pl.ds(tbase + k * TCHUNK, TCHUNK)])
        nk = k + 2
        if nk < TEXT_PER_W // TCHUNK:
            descs.append(
                pltpu.async_copy(tf.at[tidx.at[pl.ds(nk * TCHUNK, TCHUNK)]],
                                 bufs[nk % 2], sems[nk % 2]))


def _sc_body(x0s, colr, row2, wr,
             x1s, x2s, x3s,
             acc,
             colv0, colv1, colv2, wv0, wv1, wv2,
             rowv0, rowv1, rowv2, rows0, rows1, rows2,
             si0, si1, si2, sg0, sg1, sg2, ss0, ss1, ss2):
    c = lax.axis_index("c")
    s = lax.axis_index("s")

    colv = (colv0, colv1, colv2)
    wv = (wv0, wv1, wv2)
    rowv = (rowv0, rowv1, rowv2)
    rows = (rows0, rows1, rows2)
    si = (si0, si1, si2)
    sg = (sg0, sg1, sg2)
    ss = (ss0, ss1, ss2)

    coff = c * X_PAD_HALF
    ebase = s * EDGES_PER_TILE
    rbase = s * (EDGES_PER_TILE // SCHUNK)
    base = s * ROWS_PER_TILE
    zeros32 = jnp.zeros((32,), jnp.bfloat16)

    def zero_rows0():
        def zrow(i, carry):
            rows0[i, :] = zeros32
            return carry
        lax.fori_loop(0, ZCHUNK, zrow, 0)

    def zero_acc():
        def zc(k, carry):
            pltpu.sync_copy(rows0.at[pl.ds(0, ZCHUNK)],
                            acc.at[pl.ds(base + k * ZCHUNK, ZCHUNK)])
            return carry
        lax.fori_loop(0, N_ZCHUNK, zc, 0)

    zero_rows0()
    zero_acc()
    plsc.subcore_barrier()

    def start_idx(b, t):
        off = ebase + t * CHUNK
        pltpu.async_copy(colr.at[pl.ds(off, CHUNK)], colv[b], si[b])
        pltpu.async_copy(wr.at[pl.ds(2 * off, 2 * CHUNK)], wv[b], si[b])
        pltpu.async_copy(row2.at[pl.ds(rbase + t * (CHUNK // SCHUNK),
                                       CHUNK // SCHUNK)], rowv[b], si[b])

    def wait_idx(b):
        pltpu.make_async_copy(colr.at[pl.ds(0, CHUNK)], colv[b], si[b]).wait()
        pltpu.make_async_copy(wr.at[pl.ds(0, 2 * CHUNK)], wv[b], si[b]).wait()
        pltpu.make_async_copy(row2.at[pl.ds(0, CHUNK // SCHUNK)], rowv[b],
                              si[b]).wait()

    def offset(b):
        def addoff(i, carry):
            colv[b][pl.ds(i * 16, 16)] = colv[b][pl.ds(i * 16, 16)] + coff
            return carry
        lax.fori_loop(0, CHUNK // 16, addoff, 0)

    def start_g(b, src):
        for j in range(CHUNK // SCHUNK):
            pltpu.async_copy(src.at[colv[b].at[pl.ds(j * SCHUNK, SCHUNK)]],
                             rows[b].at[pl.ds(j * SCHUNK, SCHUNK)], sg[b])

    def wait_g(b, src):
        for j in range(CHUNK // SCHUNK):
            pltpu.make_async_copy(
                src.at[colv[b].at[pl.ds(j * SCHUNK, SCHUNK)]],
                rows[b].at[pl.ds(j * SCHUNK, SCHUNK)], sg[b]).wait()

    def scale(b):
        def body(i, carry):
            wvec32 = plsc.bitcast(wv[b][pl.ds(i * 32, 32)], jnp.int32)
            for j in range(16):
                e = i * 16 + j
                wsplat = plsc.bitcast(
                    jnp.full((16,), wvec32[j], jnp.int32), jnp.bfloat16)
                rows[b][e, :] = rows[b][e, :] * wsplat
            return carry
        lax.fori_loop(0, CHUNK // 16, body, 0)

    def start_sc(b):
        for j in range(CHUNK // SCHUNK):
            pltpu.async_copy(rows[b].at[pl.ds(j * SCHUNK, SCHUNK)],
                             acc.at[rowv[b].at[j]], ss[b], add=True)

    def wait_sc(b):
        for j in range(CHUNK // SCHUNK):
            pltpu.make_async_copy(rows[b].at[pl.ds(j * SCHUNK, SCHUNK)],
                                  acc.at[rowv[b].at[j]], ss[b]).wait()

    def layer(src, dst):
        start_idx(0, 0)
        start_idx(1, 1)
        start_idx(2, 2)
        wait_idx(0)
        offset(0)
        start_g(0, src)
        wait_idx(1)
        offset(1)
        start_g(1, src)

        def chunk_step(t, b, guard_first):
            bn = (b + 2) % 3
            wait_g(b, src)
            scale(b)
            start_sc(b)
            if guard_first:
                @pl.when(t >= 1)
                def _():
                    wait_sc(bn)
            else:
                wait_sc(bn)

            @pl.when(t <= LAST - 2)
            def _():
                wait_idx(bn)
                offset(bn)
                start_g(bn, src)

            @pl.when(t <= LAST - 3)
            def _():
                start_idx(b, t + 3)

        def body3(k, carry):
            for j in range(3):
                chunk_step(3 * k + j, j, j == 0)
            return carry

        lax.fori_loop(0, N_CHUNKS // 3, body3, 0)
        wait_sc(2)
        plsc.subcore_barrier()
        pltpu.sync_copy(acc.at[pl.ds(base, ROWS_PER_TILE)],
                        dst.at[pl.ds(coff + base, ROWS_PER_TILE)])
        zero_rows0()
        zero_acc()
        plsc.subcore_barrier()

    layer(x0s, x1s)
    layer(x1s, x2s)
    layer(x2s, x3s)


def _text_tc_kernel(tfg_ref, w_ref, b_ref, h_ref):
    h = jnp.dot(tfg_ref[...], w_ref[...], preferred_element_type=jnp.float32)
    h = h + b_ref[...]
    ss = jnp.sum(h * h, axis=1, keepdims=True)
    norm = jnp.maximum(jnp.sqrt(ss), 1e-12)
    h_ref[...] = h / norm


def _mean_tc_kernel(x0_ref, x1_ref, x2_ref, x3_ref, o_ref):
    acc = x0_ref[...]
    acc = acc + x1_ref[...].astype(jnp.float32)
    acc = acc + x2_ref[...].astype(jnp.float32)
    acc = acc + x3_ref[...].astype(jnp.float32)
    o_ref[...] = acc * 0.25


def kernel(user_emb, item_emb, edge_weight, text_feat, W_text, b_text,
           edge_index, items):
    row = edge_index[0].astype(jnp.int32)
    col = edge_index[1].astype(jnp.int32)
    pad = NNZ_PAD - NNZ
    colp = jnp.concatenate([col, jnp.zeros((pad,), jnp.int32)])
    wp = jnp.repeat(jnp.concatenate([edge_weight, jnp.zeros((pad,), jnp.float32)]).astype(jnp.bfloat16), 2)
    rowp = jnp.concatenate([row, jnp.zeros((pad,), jnp.int32)])
    row2 = rowp.reshape(ROW2_ROWS, SCHUNK)

    zpad = jnp.zeros((X_PAD_HALF - N_NODES, HALF), jnp.float32)
    x0s = jnp.concatenate([user_emb[:, :HALF], item_emb[:, :HALF], zpad,
                           user_emb[:, HALF:], item_emb[:, HALF:], zpad], axis=0)
    x0sb = x0s.astype(jnp.bfloat16)

    mesh = plsc.VectorSubcoreMesh(core_axis_name="c", subcore_axis_name="s")
    params = pltpu.CompilerParams(use_tc_tiling_on_sc=False,
                                  needs_layout_passes=False)

    tfg = pl.kernel(
        _text_sc_body,
        out_type=jax.ShapeDtypeStruct((N_ITEM_BATCH, TEXT_DIM), jnp.float32),
        mesh=mesh,
        compiler_params=params,
        scratch_types=[
            pltpu.VMEM((TEXT_PER_W,), jnp.int32),
            pltpu.VMEM((TCHUNK, TEXT_DIM), jnp.float32),
            pltpu.VMEM((TCHUNK, TEXT_DIM), jnp.float32),
            pltpu.SemaphoreType.DMA,
            pltpu.SemaphoreType.DMA,
        ],
    )(text_feat, items.astype(jnp.int32))

    stack_t = jax.ShapeDtypeStruct((2 * X_PAD_HALF, HALF), jnp.bfloat16)
    sc = pl.kernel(
        _sc_body,
        out_type=(stack_t, stack_t, stack_t),
        mesh=mesh,
        compiler_params=params,
        scratch_types=[
            pltpu.VMEM_SHARED((X_PAD_HALF, HALF), jnp.bfloat16),
            pltpu.VMEM((CHUNK,), jnp.int32),
            pltpu.VMEM((CHUNK,), jnp.int32),
            pltpu.VMEM((CHUNK,), jnp.int32),
            pltpu.VMEM((2 * CHUNK,), jnp.bfloat16),
            pltpu.VMEM((2 * CHUNK,), jnp.bfloat16),
            pltpu.VMEM((2 * CHUNK,), jnp.bfloat16),
            pltpu.VMEM((CHUNK // SCHUNK, SCHUNK), jnp.int32),
            pltpu.VMEM((CHUNK // SCHUNK, SCHUNK), jnp.int32),
            pltpu.VMEM((CHUNK // SCHUNK, SCHUNK), jnp.int32),
            pltpu.VMEM((CHUNK, HALF), jnp.bfloat16),
            pltpu.VMEM((CHUNK, HALF), jnp.bfloat16),
            pltpu.VMEM((CHUNK, HALF), jnp.bfloat16),
        ] + [pltpu.SemaphoreType.DMA] * 9,
    )
    x1s, x2s, x3s = sc(x0sb, colp, row2, wp)

    h = pl.pallas_call(
        _text_tc_kernel,
        grid=(N_ITEM_BATCH // 1024,),
        in_specs=[
            pl.BlockSpec((1024, TEXT_DIM), lambda i: (i, 0)),
            pl.BlockSpec((TEXT_DIM, DIM), lambda i: (0, 0)),
            pl.BlockSpec((1, DIM), lambda i: (0, 0)),
        ],
        out_specs=pl.BlockSpec((1024, DIM), lambda i: (i, 0)),
        out_shape=jax.ShapeDtypeStruct((N_ITEM_BATCH, DIM), jnp.float32),
    )(tfg, W_text, b_text.reshape(1, DIM))

    mrows = (2 * X_PAD_HALF * HALF) // 128
    resh = lambda a: a.reshape(mrows, 128)
    mblk = mrows // 8
    mspec = pl.BlockSpec((mblk, 128), lambda i: (i, 0))
    ms = pl.pallas_call(
        _mean_tc_kernel,
        grid=(8,),
        in_specs=[mspec, mspec, mspec, mspec],
        out_specs=mspec,
        out_shape=jax.ShapeDtypeStruct((mrows, 128), jnp.float32),
    )(resh(x0s), resh(x1s), resh(x2s), resh(x3s))
    ms = ms.reshape(2 * X_PAD_HALF, HALF)

    def unstack(sarr):
        return jnp.concatenate(
            [sarr[:N_NODES], sarr[X_PAD_HALF:X_PAD_HALF + N_NODES]], axis=1)

    x1 = unstack(x1s).astype(jnp.float32)
    m = unstack(ms)
    u_embedding = m[:N_USERS]
    i_embedding = m[N_USERS:]
    layer_u1 = x1[:N_USERS]
    layer_i1 = x1[N_USERS:]
    return (u_embedding, i_embedding, h, user_emb, layer_u1, item_emb, layer_i1)


# speedup vs baseline: 1.6109x; 1.6109x over previous
import jax
import jax.numpy as jnp
from jax import lax
from jax.experimental import pallas as pl
from jax.experimental.pallas import tpu as pltpu
from jax.experimental.pallas import tpu_sc as plsc

N_USERS = 25000
N_ITEMS = 25000
N_NODES = 50000
NNZ = 800000
DIM = 64
HALF = 32
TEXT_DIM = 384
N_ITEM_BATCH = 12288

N_TILES = 16
N_WORKERS = 32
SCHUNK = 128
CHUNK = 512
N_CHUNKS = 99
LAST = N_CHUNKS - 1
EDGES_PER_TILE = CHUNK * N_CHUNKS
NNZ_PAD = N_TILES * EDGES_PER_TILE
ROW2_ROWS = NNZ_PAD // SCHUNK

X_PAD_HALF = 50048
ROWS_PER_TILE = 3128
ZCHUNK = 136
N_ZCHUNK = 23
TEXT_PER_W = N_ITEM_BATCH // N_WORKERS
TCHUNK = 128


def _text_sc_body(tf, itemsr, tfg, tidx, tbuf0, tbuf1, sem0, sem1):
    c = lax.axis_index("c")
    s = lax.axis_index("s")
    wid = s * 2 + c
    tbase = wid * TEXT_PER_W
    pltpu.sync_copy(itemsr.at[pl.ds(tbase, TEXT_PER_W)], tidx)
    bufs = (tbuf0, tbuf1)
    sems = (sem0, sem1)
    descs = [
        pltpu.async_copy(tf.at[tidx.at[pl.ds(k * TCHUNK, TCHUNK)]],
                         bufs[k % 2], sems[k % 2])
        for k in range(2)
    ]
    for k in range(TEXT_PER_W // TCHUNK):
        descs[k].wait()
        pltpu.sync_copy(bufs[k % 2], tfg.at[pl.ds(tbase + k * TCHUNK, TCHUNK)])
        nk = k + 2
        if nk < TEXT_PER_W // TCHUNK:
            descs.append(
                pltpu.async_copy(tf.at[tidx.at[pl.ds(nk * TCHUNK, TCHUNK)]],
                                 bufs[nk % 2], sems[nk % 2]))


def _sc_body(x0s, colr, row2, wr,
             x1s, x2s, x3s,
             acc,
             colv0, colv1, colv2, wv0, wv1, wv2,
             rowv0, rowv1, rowv2, rows0, rows1, rows2,
             si0, si1, si2, sg0, sg1, sg2, ss0, ss1, ss2):
    c = lax.axis_index("c")
    s = lax.axis_index("s")

    colv = (colv0, colv1, colv2)
    wv = (wv0, wv1, wv2)
    rowv = (rowv0, rowv1, rowv2)
    rows = (rows0, rows1, rows2)
    si = (si0, si1, si2)
    sg = (sg0, sg1, sg2)
    ss = (ss0, ss1, ss2)

    coff = c * X_PAD_HALF
    ebase = s * EDGES_PER_TILE
    rbase = s * (EDGES_PER_TILE // SCHUNK)
    base = s * ROWS_PER_TILE
    zeros32 = jnp.zeros((32,), jnp.bfloat16)

    def zero_rows0():
        def zrow(i, carry):
            rows0[i, :] = zeros32
            return carry
        lax.fori_loop(0, ZCHUNK, zrow, 0)

    def zero_acc():
        def zc(k, carry):
            pltpu.sync_copy(rows0.at[pl.ds(0, ZCHUNK)],
                            acc.at[pl.ds(base + k * ZCHUNK, ZCHUNK)])
            return carry
        lax.fori_loop(0, N_ZCHUNK, zc, 0)

    zero_rows0()
    zero_acc()
    plsc.subcore_barrier()

    def start_idx(b, t):
        off = ebase + t * CHUNK
        pltpu.async_copy(colr.at[pl.ds(off, CHUNK)], colv[b], si[b])
        pltpu.async_copy(wr.at[pl.ds(off, CHUNK)], wv[b], si[b])
        pltpu.async_copy(row2.at[pl.ds(rbase + t * (CHUNK // SCHUNK),
                                       CHUNK // SCHUNK)], rowv[b], si[b])

    def wait_idx(b):
        pltpu.make_async_copy(colr.at[pl.ds(0, CHUNK)], colv[b], si[b]).wait()
        pltpu.make_async_copy(wr.at[pl.ds(0, CHUNK)], wv[b], si[b]).wait()
        pltpu.make_async_copy(row2.at[pl.ds(0, CHUNK // SCHUNK)], rowv[b],
                              si[b]).wait()

    def offset(b):
        def addoff(i, carry):
            colv[b][pl.ds(i * 16, 16)] = colv[b][pl.ds(i * 16, 16)] + coff
            return carry
        lax.fori_loop(0, CHUNK // 16, addoff, 0)

    def start_g(b, src):
        for j in range(CHUNK // SCHUNK):
            pltpu.async_copy(src.at[colv[b].at[pl.ds(j * SCHUNK, SCHUNK)]],
                             rows[b].at[pl.ds(j * SCHUNK, SCHUNK)], sg[b])

    def wait_g(b, src):
        for j in range(CHUNK // SCHUNK):
            pltpu.make_async_copy(
                src.at[colv[b].at[pl.ds(j * SCHUNK, SCHUNK)]],
                rows[b].at[pl.ds(j * SCHUNK, SCHUNK)], sg[b]).wait()

    def scale(b):
        def body(i, carry):
            wvec = wv[b][pl.ds(i * 16, 16)]
            wpk = plsc.pack(wvec, wvec, format=plsc.PackFormat.INTERLEAVED)
            wvec32 = plsc.bitcast(wpk, jnp.int32)
            for j in range(16):
                e = i * 16 + j
                wsplat = plsc.bitcast(
                    jnp.full((16,), wvec32[j], jnp.int32), jnp.bfloat16)
                rows[b][e, :] = rows[b][e, :] * wsplat
            return carry
        lax.fori_loop(0, CHUNK // 16, body, 0)

    def start_sc(b):
        for j in range(CHUNK // SCHUNK):
            pltpu.async_copy(rows[b].at[pl.ds(j * SCHUNK, SCHUNK)],
                             acc.at[rowv[b].at[j]], ss[b], add=True)

    def wait_sc(b):
        for j in range(CHUNK // SCHUNK):
            pltpu.make_async_copy(rows[b].at[pl.ds(j * SCHUNK, SCHUNK)],
                                  acc.at[rowv[b].at[j]], ss[b]).wait()

    def layer(src, dst):
        start_idx(0, 0)
        start_idx(1, 1)
        start_idx(2, 2)
        wait_idx(0)
        offset(0)
        start_g(0, src)
        wait_idx(1)
        offset(1)
        start_g(1, src)

        def chunk_step(t, b, guard_first):
            bn = (b + 2) % 3
            wait_g(b, src)
            scale(b)
            start_sc(b)
            if guard_first:
                @pl.when(t >= 1)
                def _():
                    wait_sc(bn)
            else:
                wait_sc(bn)

            @pl.when(t <= LAST - 2)
            def _():
                wait_idx(bn)
                offset(bn)
                start_g(bn, src)

            @pl.when(t <= LAST - 3)
            def _():
                start_idx(b, t + 3)

        def body3(k, carry):
            for j in range(3):
                chunk_step(3 * k + j, j, j == 0)
            return carry

        lax.fori_loop(0, N_CHUNKS // 3, body3, 0)
        wait_sc(2)
        plsc.subcore_barrier()
        pltpu.sync_copy(acc.at[pl.ds(base, ROWS_PER_TILE)],
                        dst.at[pl.ds(coff + base, ROWS_PER_TILE)])
        zero_rows0()
        zero_acc()
        plsc.subcore_barrier()

    layer(x0s, x1s)
    layer(x1s, x2s)
    layer(x2s, x3s)


def _text_tc_kernel(tfg_ref, w_ref, b_ref, h_ref):
    h = jnp.dot(tfg_ref[...], w_ref[...], preferred_element_type=jnp.float32)
    h = h + b_ref[...]
    ss = jnp.sum(h * h, axis=1, keepdims=True)
    norm = jnp.maximum(jnp.sqrt(ss), 1e-12)
    h_ref[...] = h / norm


def _mean_tc_kernel(x0_ref, x1_ref, x2_ref, x3_ref, o_ref, x1f_ref):
    x1f = x1_ref[...].astype(jnp.float32)
    x1f_ref[...] = x1f
    acc = x0_ref[...] + x1f
    acc = acc + x2_ref[...].astype(jnp.float32)
    acc = acc + x3_ref[...].astype(jnp.float32)
    o_ref[...] = acc * 0.25


def kernel(user_emb, item_emb, edge_weight, text_feat, W_text, b_text,
           edge_index, items):
    row = edge_index[0].astype(jnp.int32)
    col = edge_index[1].astype(jnp.int32)
    pad = NNZ_PAD - NNZ
    colp = jnp.concatenate([col, jnp.zeros((pad,), jnp.int32)])
    wp = jnp.concatenate([edge_weight, jnp.zeros((pad,), jnp.float32)])
    rowp = jnp.concatenate([row, jnp.zeros((pad,), jnp.int32)])
    row2 = rowp.reshape(ROW2_ROWS, SCHUNK)

    zpad = jnp.zeros((X_PAD_HALF - N_NODES, HALF), jnp.float32)
    x0s = jnp.concatenate([user_emb[:, :HALF], item_emb[:, :HALF], zpad,
                           user_emb[:, HALF:], item_emb[:, HALF:], zpad], axis=0)
    x0sb = x0s.astype(jnp.bfloat16)

    mesh = plsc.VectorSubcoreMesh(core_axis_name="c", subcore_axis_name="s")
    params = pltpu.CompilerParams(use_tc_tiling_on_sc=False,
                                  needs_layout_passes=False)

    tfg = pl.kernel(
        _text_sc_body,
        out_type=jax.ShapeDtypeStruct((N_ITEM_BATCH, TEXT_DIM), jnp.float32),
        mesh=mesh,
        compiler_params=params,
        scratch_types=[
            pltpu.VMEM((TEXT_PER_W,), jnp.int32),
            pltpu.VMEM((TCHUNK, TEXT_DIM), jnp.float32),
            pltpu.VMEM((TCHUNK, TEXT_DIM), jnp.float32),
            pltpu.SemaphoreType.DMA,
            pltpu.SemaphoreType.DMA,
        ],
    )(text_feat, items.astype(jnp.int32))

    stack_t = jax.ShapeDtypeStruct((2 * X_PAD_HALF, HALF), jnp.bfloat16)
    sc = pl.kernel(
        _sc_body,
        out_type=(stack_t, stack_t, stack_t),
        mesh=mesh,
        compiler_params=params,
        scratch_types=[
            pltpu.VMEM_SHARED((X_PAD_HALF, HALF), jnp.bfloat16),
            pltpu.VMEM((CHUNK,), jnp.int32),
            pltpu.VMEM((CHUNK,), jnp.int32),
            pltpu.VMEM((CHUNK,), jnp.int32),
            pltpu.VMEM((CHUNK,), jnp.float32),
            pltpu.VMEM((CHUNK,), jnp.float32),
            pltpu.VMEM((CHUNK,), jnp.float32),
            pltpu.VMEM((CHUNK // SCHUNK, SCHUNK), jnp.int32),
            pltpu.VMEM((CHUNK // SCHUNK, SCHUNK), jnp.int32),
            pltpu.VMEM((CHUNK // SCHUNK, SCHUNK), jnp.int32),
            pltpu.VMEM((CHUNK, HALF), jnp.bfloat16),
            pltpu.VMEM((CHUNK, HALF), jnp.bfloat16),
            pltpu.VMEM((CHUNK, HALF), jnp.bfloat16),
        ] + [pltpu.SemaphoreType.DMA] * 9,
    )
    x1s, x2s, x3s = sc(x0sb, colp, row2, wp)

    h = pl.pallas_call(
        _text_tc_kernel,
        grid=(N_ITEM_BATCH // 1024,),
        in_specs=[
            pl.BlockSpec((1024, TEXT_DIM), lambda i: (i, 0)),
            pl.BlockSpec((TEXT_DIM, DIM), lambda i: (0, 0)),
            pl.BlockSpec((1, DIM), lambda i: (0, 0)),
        ],
        out_specs=pl.BlockSpec((1024, DIM), lambda i: (i, 0)),
        out_shape=jax.ShapeDtypeStruct((N_ITEM_BATCH, DIM), jnp.float32),
    )(tfg, W_text, b_text.reshape(1, DIM))

    mrows = (2 * X_PAD_HALF * HALF) // 128
    resh = lambda a: a.reshape(mrows, 128)
    mblk = mrows // 8
    mspec = pl.BlockSpec((mblk, 128), lambda i: (i, 0))
    ms = pl.pallas_call(
        _mean_tc_kernel,
        grid=(8,),
        in_specs=[mspec, mspec, mspec, mspec],
        out_specs=(mspec, mspec),
        out_shape=(jax.ShapeDtypeStruct((mrows, 128), jnp.float32),
                   jax.ShapeDtypeStruct((mrows, 128), jnp.float32)),
    )(resh(x0s), resh(x1s), resh(x2s), resh(x3s))
    ms, x1f = ms
    ms = ms.reshape(2 * X_PAD_HALF, HALF)
    x1f = x1f.reshape(2 * X_PAD_HALF, HALF)

    def unstack(sarr):
        return jnp.concatenate(
            [sarr[:N_NODES], sarr[X_PAD_HALF:X_PAD_HALF + N_NODES]], axis=1)

    x1 = unstack(x1f)
    m = unstack(ms)
    u_embedding = m[:N_USERS]
    i_embedding = m[N_USERS:]
    layer_u1 = x1[:N_USERS]
    layer_i1 = x1[N_USERS:]
    return (u_embedding, i_embedding, h, user_emb, layer_u1, item_emb, layer_i1)


# speedup vs baseline: 1.7075x; 1.0600x over previous
import jax
import jax.numpy as jnp
from jax import lax
from jax.experimental import pallas as pl
from jax.experimental.pallas import tpu as pltpu
from jax.experimental.pallas import tpu_sc as plsc

N_USERS = 25000
N_ITEMS = 25000
N_NODES = 50000
NNZ = 800000
DIM = 64
HALF = 32
TEXT_DIM = 384
N_ITEM_BATCH = 12288

N_TILES = 16
N_WORKERS = 32
SCHUNK = 128
CHUNK = 512
N_CHUNKS = 99
LAST = N_CHUNKS - 1
EDGES_PER_TILE = CHUNK * N_CHUNKS
NNZ_PAD = N_TILES * EDGES_PER_TILE
ROW2_ROWS = NNZ_PAD // SCHUNK

X_PAD_HALF = 50048
ROWS_PER_TILE = 3128
ZCHUNK = 136
N_ZCHUNK = 23
TEXT_PER_W = N_ITEM_BATCH // N_WORKERS
TCHUNK = 128


def _text_sc_body(tf, itemsr, tfg, tidx, tbuf0, tbuf1, sem0, sem1):
    c = lax.axis_index("c")
    s = lax.axis_index("s")
    wid = s * 2 + c
    tbase = wid * TEXT_PER_W
    pltpu.sync_copy(itemsr.at[pl.ds(tbase, TEXT_PER_W)], tidx)
    bufs = (tbuf0, tbuf1)
    sems = (sem0, sem1)
    descs = [
        pltpu.async_copy(tf.at[tidx.at[pl.ds(k * TCHUNK, TCHUNK)]],
                         bufs[k % 2], sems[k % 2])
        for k in range(2)
    ]
    for k in range(TEXT_PER_W // TCHUNK):
        descs[k].wait()
        pltpu.sync_copy(bufs[k % 2], tfg.at[pl.ds(tbase + k * TCHUNK, TCHUNK)])
        nk = k + 2
        if nk < TEXT_PER_W // TCHUNK:
            descs.append(
                pltpu.async_copy(tf.at[tidx.at[pl.ds(nk * TCHUNK, TCHUNK)]],
                                 bufs[nk % 2], sems[nk % 2]))


def _sc_body(x0s, cwr, row2,
             x1s, x2s, x3s,
             acc,
             colv0, colv1, colv2, wv0, wv1, wv2,
             rowv0, rowv1, rowv2, rows0, rows1, rows2,
             si0, si1, si2, sg0, sg1, sg2, ss0, ss1, ss2):
    c = lax.axis_index("c")
    s = lax.axis_index("s")

    colv = (colv0, colv1, colv2)
    wv = (wv0, wv1, wv2)
    rowv = (rowv0, rowv1, rowv2)
    rows = (rows0, rows1, rows2)
    si = (si0, si1, si2)
    sg = (sg0, sg1, sg2)
    ss = (ss0, ss1, ss2)

    coff = c * X_PAD_HALF
    ebase = s * EDGES_PER_TILE
    rbase = s * (EDGES_PER_TILE // SCHUNK)
    base = s * ROWS_PER_TILE
    zeros32 = jnp.zeros((32,), jnp.bfloat16)

    def zero_rows0():
        def zrow(i, carry):
            rows0[i, :] = zeros32
            return carry
        lax.fori_loop(0, ZCHUNK, zrow, 0)

    def zero_acc():
        def zc(k, carry):
            pltpu.sync_copy(rows0.at[pl.ds(0, ZCHUNK)],
                            acc.at[pl.ds(base + k * ZCHUNK, ZCHUNK)])
            return carry
        lax.fori_loop(0, N_ZCHUNK, zc, 0)

    zero_rows0()
    zero_acc()
    plsc.subcore_barrier()

    def start_idx(b, t):
        off = ebase + t * CHUNK
        pltpu.async_copy(cwr.at[pl.ds(off, CHUNK)], colv[b], si[b])
        pltpu.async_copy(cwr.at[pl.ds(NNZ_PAD + off, CHUNK)], wv[b], si[b])
        pltpu.async_copy(row2.at[pl.ds(rbase + t * (CHUNK // SCHUNK),
                                       CHUNK // SCHUNK)], rowv[b], si[b])

    def wait_idx(b):
        pltpu.make_async_copy(cwr.at[pl.ds(0, CHUNK)], colv[b], si[b]).wait()
        pltpu.make_async_copy(cwr.at[pl.ds(0, CHUNK)], wv[b], si[b]).wait()
        pltpu.make_async_copy(row2.at[pl.ds(0, CHUNK // SCHUNK)], rowv[b],
                              si[b]).wait()

    def offset(b):
        def addoff(i, carry):
            colv[b][pl.ds(i * 16, 16)] = colv[b][pl.ds(i * 16, 16)] + coff
            return carry
        lax.fori_loop(0, CHUNK // 16, addoff, 0)

    def start_g(b, src):
        for j in range(CHUNK // SCHUNK):
            pltpu.async_copy(src.at[colv[b].at[pl.ds(j * SCHUNK, SCHUNK)]],
                             rows[b].at[pl.ds(j * SCHUNK, SCHUNK)], sg[b])

    def wait_g(b, src):
        for j in range(CHUNK // SCHUNK):
            pltpu.make_async_copy(
                src.at[colv[b].at[pl.ds(j * SCHUNK, SCHUNK)]],
                rows[b].at[pl.ds(j * SCHUNK, SCHUNK)], sg[b]).wait()

    def scale(b):
        def body(i, carry):
            wvec = plsc.bitcast(wv[b][pl.ds(i * 16, 16)], jnp.float32)
            wpk = plsc.pack(wvec, wvec, format=plsc.PackFormat.INTERLEAVED)
            wvec32 = plsc.bitcast(wpk, jnp.int32)
            for j in range(16):
                e = i * 16 + j
                wsplat = plsc.bitcast(
                    jnp.full((16,), wvec32[j], jnp.int32), jnp.bfloat16)
                rows[b][e, :] = rows[b][e, :] * wsplat
            return carry
        lax.fori_loop(0, CHUNK // 16, body, 0)

    def start_sc(b):
        for j in range(CHUNK // SCHUNK):
            pltpu.async_copy(rows[b].at[pl.ds(j * SCHUNK, SCHUNK)],
                             acc.at[rowv[b].at[j]], ss[b], add=True)

    def wait_sc(b):
        for j in range(CHUNK // SCHUNK):
            pltpu.make_async_copy(rows[b].at[pl.ds(j * SCHUNK, SCHUNK)],
                                  acc.at[rowv[b].at[j]], ss[b]).wait()

    def layer(src, dst):
        start_idx(0, 0)
        start_idx(1, 1)
        start_idx(2, 2)
        wait_idx(0)
        offset(0)
        start_g(0, src)
        wait_idx(1)
        offset(1)
        start_g(1, src)

        def chunk_step(t, b, guard_first):
            bn = (b + 2) % 3
            wait_g(b, src)
            scale(b)
            start_sc(b)
            if guard_first:
                @pl.when(t >= 1)
                def _():
                    wait_sc(bn)
            else:
                wait_sc(bn)

            @pl.when(t <= LAST - 2)
            def _():
                wait_idx(bn)
                offset(bn)
                start_g(bn, src)

            @pl.when(t <= LAST - 3)
            def _():
                start_idx(b, t + 3)

        def body3(k, carry):
            for j in range(3):
                chunk_step(3 * k + j, j, j == 0)
            return carry

        lax.fori_loop(0, N_CHUNKS // 3, body3, 0)
        wait_sc(2)
        plsc.subcore_barrier()
        pltpu.sync_copy(acc.at[pl.ds(base, ROWS_PER_TILE)],
                        dst.at[pl.ds(coff + base, ROWS_PER_TILE)])
        zero_rows0()
        zero_acc()
        plsc.subcore_barrier()

    layer(x0s, x1s)
    layer(x1s, x2s)
    layer(x2s, x3s)


def _text_tc_kernel(tfg_ref, w_ref, b_ref, h_ref):
    h = jnp.dot(tfg_ref[...], w_ref[...], preferred_element_type=jnp.float32)
    h = h + b_ref[...]
    ss = jnp.sum(h * h, axis=1, keepdims=True)
    norm = jnp.maximum(jnp.sqrt(ss), 1e-12)
    h_ref[...] = h / norm


def _mean_tc_kernel(x0l, x0h, x1l, x1h, x2l, x2h, x3l, x3h, o_ref, x1f_ref):
    for half, (x0, x1, x2, x3) in enumerate(
            ((x0l, x1l, x2l, x3l), (x0h, x1h, x2h, x3h))):
        x1f = x1[...].astype(jnp.float32)
        sl = (slice(None), slice(half * HALF, (half + 1) * HALF))
        x1f_ref[sl] = x1f
        acc = x0[...] + x1f
        acc = acc + x2[...].astype(jnp.float32)
        acc = acc + x3[...].astype(jnp.float32)
        o_ref[sl] = acc * 0.25


def kernel(user_emb, item_emb, edge_weight, text_feat, W_text, b_text,
           edge_index, items):
    row = edge_index[0].astype(jnp.int32)
    col = edge_index[1].astype(jnp.int32)
    pad = NNZ_PAD - NNZ
    zpi = jnp.zeros((pad,), jnp.int32)
    cw = jnp.concatenate([col, zpi, lax.bitcast_convert_type(edge_weight, jnp.int32), zpi])
    row2 = jnp.concatenate([row, zpi]).reshape(ROW2_ROWS, SCHUNK)

    zpad = jnp.zeros((X_PAD_HALF - N_NODES, HALF), jnp.float32)
    x0s = jnp.concatenate([user_emb[:, :HALF], item_emb[:, :HALF], zpad,
                           user_emb[:, HALF:], item_emb[:, HALF:], zpad], axis=0)
    x0sb = x0s.astype(jnp.bfloat16)

    mesh = plsc.VectorSubcoreMesh(core_axis_name="c", subcore_axis_name="s")
    params = pltpu.CompilerParams(use_tc_tiling_on_sc=False,
                                  needs_layout_passes=False)

    tfg = pl.kernel(
        _text_sc_body,
        out_type=jax.ShapeDtypeStruct((N_ITEM_BATCH, TEXT_DIM), jnp.float32),
        mesh=mesh,
        compiler_params=params,
        scratch_types=[
            pltpu.VMEM((TEXT_PER_W,), jnp.int32),
            pltpu.VMEM((TCHUNK, TEXT_DIM), jnp.float32),
            pltpu.VMEM((TCHUNK, TEXT_DIM), jnp.float32),
            pltpu.SemaphoreType.DMA,
            pltpu.SemaphoreType.DMA,
        ],
    )(text_feat, items.astype(jnp.int32))

    stack_t = jax.ShapeDtypeStruct((2 * X_PAD_HALF, HALF), jnp.bfloat16)
    sc = pl.kernel(
        _sc_body,
        out_type=(stack_t, stack_t, stack_t),
        mesh=mesh,
        compiler_params=params,
        scratch_types=[
            pltpu.VMEM_SHARED((X_PAD_HALF, HALF), jnp.bfloat16),
            pltpu.VMEM((CHUNK,), jnp.int32),
            pltpu.VMEM((CHUNK,), jnp.int32),
            pltpu.VMEM((CHUNK,), jnp.int32),
            pltpu.VMEM((CHUNK,), jnp.int32),
            pltpu.VMEM((CHUNK,), jnp.int32),
            pltpu.VMEM((CHUNK,), jnp.int32),
            pltpu.VMEM((CHUNK // SCHUNK, SCHUNK), jnp.int32),
            pltpu.VMEM((CHUNK // SCHUNK, SCHUNK), jnp.int32),
            pltpu.VMEM((CHUNK // SCHUNK, SCHUNK), jnp.int32),
            pltpu.VMEM((CHUNK, HALF), jnp.bfloat16),
            pltpu.VMEM((CHUNK, HALF), jnp.bfloat16),
            pltpu.VMEM((CHUNK, HALF), jnp.bfloat16),
        ] + [pltpu.SemaphoreType.DMA] * 9,
    )
    x1s, x2s, x3s = sc(x0sb, cw, row2)

    h = pl.pallas_call(
        _text_tc_kernel,
        grid=(N_ITEM_BATCH // 1024,),
        in_specs=[
            pl.BlockSpec((1024, TEXT_DIM), lambda i: (i, 0)),
            pl.BlockSpec((TEXT_DIM, DIM), lambda i: (0, 0)),
            pl.BlockSpec((1, DIM), lambda i: (0, 0)),
        ],
        out_specs=pl.BlockSpec((1024, DIM), lambda i: (i, 0)),
        out_shape=jax.ShapeDtypeStruct((N_ITEM_BATCH, DIM), jnp.float32),
    )(tfg, W_text, b_text.reshape(1, DIM))

    MB = ROWS_PER_TILE
    lo = pl.BlockSpec((MB, HALF), lambda i: (i, 0))
    hi = pl.BlockSpec((MB, HALF), lambda i: (i + N_TILES, 0))
    ospec = pl.BlockSpec((MB, DIM), lambda i: (i, 0))
    m, x1 = pl.pallas_call(
        _mean_tc_kernel,
        grid=(N_TILES,),
        in_specs=[lo, hi, lo, hi, lo, hi, lo, hi],
        out_specs=(ospec, ospec),
        out_shape=(jax.ShapeDtypeStruct((X_PAD_HALF, DIM), jnp.float32),
                   jax.ShapeDtypeStruct((X_PAD_HALF, DIM), jnp.float32)),
    )(x0s, x0s, x1s, x1s, x2s, x2s, x3s, x3s)
    u_embedding = m[:N_USERS]
    i_embedding = m[N_USERS:N_NODES]
    layer_u1 = x1[:N_USERS]
    layer_i1 = x1[N_USERS:N_NODES]
    return (u_embedding, i_embedding, h, user_emb, layer_u1, item_emb, layer_i1)


# speedup vs baseline: 1.8017x; 1.0552x over previous
import jax
import jax.numpy as jnp
from jax import lax
from jax.experimental import pallas as pl
from jax.experimental.pallas import tpu as pltpu
from jax.experimental.pallas import tpu_sc as plsc

N_USERS = 25000
N_ITEMS = 25000
N_NODES = 50000
NNZ = 800000
DIM = 64
HALF = 32
TEXT_DIM = 384
N_ITEM_BATCH = 12288

N_TILES = 16
N_WORKERS = 32
SCHUNK = 128
CHUNK = 512
N_CHUNKS = 99
LAST = N_CHUNKS - 1
EDGES_PER_TILE = CHUNK * N_CHUNKS
NNZ_PAD = N_TILES * EDGES_PER_TILE
ROW2_ROWS = NNZ_PAD // SCHUNK

X_PAD_HALF = 50000
ROWS_PER_TILE = 3125
ZCHUNK = 125
N_ZCHUNK = 25
TEXT_PER_W = N_ITEM_BATCH // N_WORKERS
TCHUNK = 128


def _text_sc_body(tf, itemsr, tfg, tidx, tbuf0, tbuf1, sem0, sem1):
    c = lax.axis_index("c")
    s = lax.axis_index("s")
    wid = s * 2 + c
    tbase = wid * TEXT_PER_W
    pltpu.sync_copy(itemsr.at[pl.ds(tbase, TEXT_PER_W)], tidx)
    bufs = (tbuf0, tbuf1)
    sems = (sem0, sem1)
    descs = [
        pltpu.async_copy(tf.at[tidx.at[pl.ds(k * TCHUNK, TCHUNK)]],
                         bufs[k % 2], sems[k % 2])
        for k in range(2)
    ]
    for k in range(TEXT_PER_W // TCHUNK):
        descs[k].wait()
        pltpu.sync_copy(bufs[k % 2], tfg.at[pl.ds(tbase + k * TCHUNK, TCHUNK)])
        nk = k + 2
        if nk < TEXT_PER_W // TCHUNK:
            descs.append(
                pltpu.async_copy(tf.at[tidx.at[pl.ds(nk * TCHUNK, TCHUNK)]],
                                 bufs[nk % 2], sems[nk % 2]))


def _sc_body(x0s, cwr, row2,
             x1s, x2s, x3s,
             acc,
             colv0, colv1, colv2, wv0, wv1, wv2,
             rowv0, rowv1, rowv2, rows0, rows1, rows2,
             si0, si1, si2, sg0, sg1, sg2, ss0, ss1, ss2):
    c = lax.axis_index("c")
    s = lax.axis_index("s")

    colv = (colv0, colv1, colv2)
    wv = (wv0, wv1, wv2)
    rowv = (rowv0, rowv1, rowv2)
    rows = (rows0, rows1, rows2)
    si = (si0, si1, si2)
    sg = (sg0, sg1, sg2)
    ss = (ss0, ss1, ss2)

    coff = c * X_PAD_HALF
    ebase = s * EDGES_PER_TILE
    rbase = s * (EDGES_PER_TILE // SCHUNK)
    base = s * ROWS_PER_TILE
    zeros32 = jnp.zeros((32,), jnp.bfloat16)

    def zero_rows0():
        def zrow(i, carry):
            rows0[i, :] = zeros32
            return carry
        lax.fori_loop(0, ZCHUNK, zrow, 0)

    def zero_acc():
        def zc(k, carry):
            pltpu.sync_copy(rows0.at[pl.ds(0, ZCHUNK)],
                            acc.at[pl.ds(base + k * ZCHUNK, ZCHUNK)])
            return carry
        lax.fori_loop(0, N_ZCHUNK, zc, 0)

    zero_rows0()
    zero_acc()
    plsc.subcore_barrier()

    def start_idx(b, t):
        off = ebase + t * CHUNK
        pltpu.async_copy(cwr.at[pl.ds(off, CHUNK)], colv[b], si[b])
        pltpu.async_copy(cwr.at[pl.ds(NNZ_PAD + off, CHUNK)], wv[b], si[b])
        pltpu.async_copy(row2.at[pl.ds(rbase + t * (CHUNK // SCHUNK),
                                       CHUNK // SCHUNK)], rowv[b], si[b])

    def wait_idx(b):
        pltpu.make_async_copy(cwr.at[pl.ds(0, CHUNK)], colv[b], si[b]).wait()
        pltpu.make_async_copy(cwr.at[pl.ds(0, CHUNK)], wv[b], si[b]).wait()
        pltpu.make_async_copy(row2.at[pl.ds(0, CHUNK // SCHUNK)], rowv[b],
                              si[b]).wait()

    def offset(b):
        def addoff(i, carry):
            colv[b][pl.ds(i * 16, 16)] = colv[b][pl.ds(i * 16, 16)] + coff
            return carry
        lax.fori_loop(0, CHUNK // 16, addoff, 0)

    def start_g(b, src):
        for j in range(CHUNK // SCHUNK):
            pltpu.async_copy(src.at[colv[b].at[pl.ds(j * SCHUNK, SCHUNK)]],
                             rows[b].at[pl.ds(j * SCHUNK, SCHUNK)], sg[b])

    def wait_g(b, src):
        for j in range(CHUNK // SCHUNK):
            pltpu.make_async_copy(
                src.at[colv[b].at[pl.ds(j * SCHUNK, SCHUNK)]],
                rows[b].at[pl.ds(j * SCHUNK, SCHUNK)], sg[b]).wait()

    def scale(b):
        def body(i, carry):
            wvec = plsc.bitcast(wv[b][pl.ds(i * 16, 16)], jnp.float32)
            wpk = plsc.pack(wvec, wvec, format=plsc.PackFormat.INTERLEAVED)
            wvec32 = plsc.bitcast(wpk, jnp.int32)
            for j in range(16):
                e = i * 16 + j
                wsplat = plsc.bitcast(
                    jnp.full((16,), wvec32[j], jnp.int32), jnp.bfloat16)
                rows[b][e, :] = rows[b][e, :] * wsplat
            return carry
        lax.fori_loop(0, CHUNK // 16, body, 0)

    def start_sc(b):
        for j in range(CHUNK // SCHUNK):
            pltpu.async_copy(rows[b].at[pl.ds(j * SCHUNK, SCHUNK)],
                             acc.at[rowv[b].at[j]], ss[b], add=True)

    def wait_sc(b):
        for j in range(CHUNK // SCHUNK):
            pltpu.make_async_copy(rows[b].at[pl.ds(j * SCHUNK, SCHUNK)],
                                  acc.at[rowv[b].at[j]], ss[b]).wait()

    def layer(src, dst):
        start_idx(0, 0)
        start_idx(1, 1)
        start_idx(2, 2)
        wait_idx(0)
        offset(0)
        start_g(0, src)
        wait_idx(1)
        offset(1)
        start_g(1, src)

        def chunk_step(t, b, guard_first):
            bn = (b + 2) % 3
            wait_g(b, src)
            scale(b)
            start_sc(b)
            if guard_first:
                @pl.when(t >= 1)
                def _():
                    wait_sc(bn)
            else:
                wait_sc(bn)

            @pl.when(t <= LAST - 2)
            def _():
                wait_idx(bn)
                offset(bn)
                start_g(bn, src)

            @pl.when(t <= LAST - 3)
            def _():
                start_idx(b, t + 3)

        def body3(k, carry):
            for j in range(3):
                chunk_step(3 * k + j, j, j == 0)
            return carry

        lax.fori_loop(0, N_CHUNKS // 3, body3, 0)
        wait_sc(2)
        plsc.subcore_barrier()
        pltpu.sync_copy(acc.at[pl.ds(base, ROWS_PER_TILE)],
                        dst.at[pl.ds(coff + base, ROWS_PER_TILE)])
        zero_rows0()
        zero_acc()
        plsc.subcore_barrier()

    layer(x0s, x1s)
    layer(x1s, x2s)
    layer(x2s, x3s)


def _text_tc_kernel(tfg_ref, w_ref, b_ref, h_ref):
    h = jnp.dot(tfg_ref[...], w_ref[...], preferred_element_type=jnp.float32)
    h = h + b_ref[...]
    ss = jnp.sum(h * h, axis=1, keepdims=True)
    norm = jnp.maximum(jnp.sqrt(ss), 1e-12)
    h_ref[...] = h / norm


def _mean_tc_kernel(x0, x1l, x1h, x2l, x2h, x3l, x3h, o_ref, x1f_ref):
    x0v = x0[...]
    for half, (x1, x2, x3) in enumerate(
            ((x1l, x2l, x3l), (x1h, x2h, x3h))):
        x1f = x1[...].astype(jnp.float32)
        sl = (slice(None), slice(half * HALF, (half + 1) * HALF))
        x1f_ref[sl] = x1f
        acc = x0v[sl] + x1f
        acc = acc + x2[...].astype(jnp.float32)
        acc = acc + x3[...].astype(jnp.float32)
        o_ref[sl] = acc * 0.25


def kernel(user_emb, item_emb, edge_weight, text_feat, W_text, b_text,
           edge_index, items):
    row = edge_index[0].astype(jnp.int32)
    col = edge_index[1].astype(jnp.int32)
    pad = NNZ_PAD - NNZ
    zpi = jnp.zeros((pad,), jnp.int32)
    cw = jnp.concatenate([col, zpi, lax.bitcast_convert_type(edge_weight, jnp.int32), zpi])
    row2 = jnp.concatenate([row, zpi]).reshape(ROW2_ROWS, SCHUNK)

    x0sb = jnp.concatenate(
        [user_emb[:, :HALF], item_emb[:, :HALF],
         user_emb[:, HALF:], item_emb[:, HALF:]], axis=0).astype(jnp.bfloat16)

    mesh = plsc.VectorSubcoreMesh(core_axis_name="c", subcore_axis_name="s")
    params = pltpu.CompilerParams(use_tc_tiling_on_sc=False,
                                  needs_layout_passes=False)

    tfg = pl.kernel(
        _text_sc_body,
        out_type=jax.ShapeDtypeStruct((N_ITEM_BATCH, TEXT_DIM), jnp.float32),
        mesh=mesh,
        compiler_params=params,
        scratch_types=[
            pltpu.VMEM((TEXT_PER_W,), jnp.int32),
            pltpu.VMEM((TCHUNK, TEXT_DIM), jnp.float32),
            pltpu.VMEM((TCHUNK, TEXT_DIM), jnp.float32),
            pltpu.SemaphoreType.DMA,
            pltpu.SemaphoreType.DMA,
        ],
    )(text_feat, items.astype(jnp.int32))

    stack_t = jax.ShapeDtypeStruct((2 * X_PAD_HALF, HALF), jnp.bfloat16)
    sc = pl.kernel(
        _sc_body,
        out_type=(stack_t, stack_t, stack_t),
        mesh=mesh,
        compiler_params=params,
        scratch_types=[
            pltpu.VMEM_SHARED((X_PAD_HALF, HALF), jnp.bfloat16),
            pltpu.VMEM((CHUNK,), jnp.int32),
            pltpu.VMEM((CHUNK,), jnp.int32),
            pltpu.VMEM((CHUNK,), jnp.int32),
            pltpu.VMEM((CHUNK,), jnp.int32),
            pltpu.VMEM((CHUNK,), jnp.int32),
            pltpu.VMEM((CHUNK,), jnp.int32),
            pltpu.VMEM((CHUNK // SCHUNK, SCHUNK), jnp.int32),
            pltpu.VMEM((CHUNK // SCHUNK, SCHUNK), jnp.int32),
            pltpu.VMEM((CHUNK // SCHUNK, SCHUNK), jnp.int32),
            pltpu.VMEM((CHUNK, HALF), jnp.bfloat16),
            pltpu.VMEM((CHUNK, HALF), jnp.bfloat16),
            pltpu.VMEM((CHUNK, HALF), jnp.bfloat16),
        ] + [pltpu.SemaphoreType.DMA] * 9,
    )
    x1s, x2s, x3s = sc(x0sb, cw, row2)

    h = pl.pallas_call(
        _text_tc_kernel,
        grid=(N_ITEM_BATCH // 1024,),
        in_specs=[
            pl.BlockSpec((1024, TEXT_DIM), lambda i: (i, 0)),
            pl.BlockSpec((TEXT_DIM, DIM), lambda i: (0, 0)),
            pl.BlockSpec((1, DIM), lambda i: (0, 0)),
        ],
        out_specs=pl.BlockSpec((1024, DIM), lambda i: (i, 0)),
        out_shape=jax.ShapeDtypeStruct((N_ITEM_BATCH, DIM), jnp.float32),
    )(tfg, W_text, b_text.reshape(1, DIM))

    MB = 1000
    NBLK = N_USERS // MB

    def mean_call(x0_full, off):
        x0spec = pl.BlockSpec((MB, DIM), lambda i: (i, 0))
        lo = pl.BlockSpec((MB, HALF), lambda i: (i + off, 0))
        hi = pl.BlockSpec((MB, HALF), lambda i: (i + off + 2 * NBLK, 0))
        ospec = pl.BlockSpec((MB, DIM), lambda i: (i, 0))
        return pl.pallas_call(
            _mean_tc_kernel,
            grid=(NBLK,),
            in_specs=[x0spec, lo, hi, lo, hi, lo, hi],
            out_specs=(ospec, ospec),
            out_shape=(jax.ShapeDtypeStruct((N_USERS, DIM), jnp.float32),
                       jax.ShapeDtypeStruct((N_USERS, DIM), jnp.float32)),
        )(x0_full, x1s, x1s, x2s, x2s, x3s, x3s)

    u_embedding, layer_u1 = mean_call(user_emb, 0)
    i_embedding, layer_i1 = mean_call(item_emb, NBLK)
    return (u_embedding, i_embedding, h, user_emb, layer_u1, item_emb, layer_i1)
